# baseline (speedup 1.0000x reference)
import functools

import jax
import jax.numpy as jnp
from jax import lax
from jax.experimental import pallas as pl
from jax.experimental.pallas import tpu as pltpu
from jax.experimental.pallas import tpu_sc as plsc

N = 10000
D = 128
E = 320000
NC = 2
NS = 16
NW = NC * NS
EPT = E // NW
B = 80
NB = EPT // B
NCHUNK_P = N // B
KMAX_P = -(-NCHUNK_P // NS)

R = 400
G = N // R

_mesh = plsc.VectorSubcoreMesh(
    core_axis_name="c", subcore_axis_name="s", num_cores=NC, num_subcores=NS
)


@functools.partial(
    pl.kernel,
    out_type=jax.ShapeDtypeStruct((NC, N, D), jnp.float32),
    mesh=_mesh,
    scratch_types=[
        pltpu.VMEM_SHARED((N, D), jnp.float32),
        pltpu.VMEM((EPT,), jnp.int32),
        pltpu.VMEM((2, B, D), jnp.float32),
        pltpu.SemaphoreType.DMA,
        pltpu.SemaphoreType.DMA,
    ],
)
def _deg_kernel(dst_hbm, zeros_hbm, ones_hbm, deg_out,
                deg_sp, dst_v, rows2_v, sem_a, sem_b):
    c = lax.axis_index("c")
    sid = lax.axis_index("s")
    wid = sid * NC + c
    zeros_v = rows2_v.at[0]
    ones_v = rows2_v.at[1]
    pltpu.sync_copy(zeros_hbm, zeros_v)
    pltpu.sync_copy(ones_hbm, ones_v)
    pltpu.sync_copy(dst_hbm.at[wid], dst_v)

    def _zc(k):
        cidx = sid + k * NS
        return pltpu.make_async_copy(
            zeros_v, deg_sp.at[pl.ds(cidx * B, B)], sem_a
        )

    for k in range(KMAX_P):
        @pl.when(sid + k * NS < NCHUNK_P)
        def _(k=k):
            _zc(k).start()

    for k in range(KMAX_P):
        @pl.when(sid + k * NS < NCHUNK_P)
        def _(k=k):
            _zc(k).wait()

    plsc.subcore_barrier()

    def _sc(bi):
        return pltpu.make_async_copy(
            ones_v, deg_sp.at[dst_v.at[pl.ds(bi * B, B)]], sem_a
        )

    def fire(bi, carry):
        _sc(bi).start(add=True)
        return carry

    def drain(bi, carry):
        _sc(bi).wait()
        return carry

    lax.fori_loop(0, NB, fire, 0)
    lax.fori_loop(0, NB, drain, 0)
    plsc.subcore_barrier()

    def _stage(k, j):
        cidx = sid + k * NS
        return pltpu.make_async_copy(
            deg_sp.at[pl.ds(cidx * B, B)], rows2_v.at[j], sem_a
        )

    def _wr(k, j):
        cidx = sid + k * NS
        return pltpu.make_async_copy(
            rows2_v.at[j], deg_out.at[c, pl.ds(cidx * B, B)], sem_b
        )

    for k in range(KMAX_P):
        if k >= 2:
            @pl.when(sid + (k - 2) * NS < NCHUNK_P)
            def _(k=k):
                _wr(k - 2, k % 2).wait()

        @pl.when(sid + k * NS < NCHUNK_P)
        def _(k=k):
            _stage(k, k % 2).start()
            _stage(k, k % 2).wait()
            _wr(k, k % 2).start()

    for k in range(max(0, KMAX_P - 2), KMAX_P):
        @pl.when(sid + k * NS < NCHUNK_P)
        def _(k=k):
            _wr(k, k % 2).wait()


@functools.partial(
    pl.kernel,
    out_type=jax.ShapeDtypeStruct((NC, N, D), jnp.float32),
    mesh=_mesh,
    scratch_types=[
        pltpu.VMEM_SHARED((N, D), jnp.float32),
        pltpu.VMEM((EPT,), jnp.int32),
        pltpu.VMEM((EPT,), jnp.int32),
        pltpu.VMEM((3, B, D), jnp.float32),
        pltpu.SemaphoreType.DMA,
        pltpu.SemaphoreType.DMA,
        pltpu.SemaphoreType.DMA,
        pltpu.SemaphoreType.DMA,
        pltpu.SemaphoreType.DMA,
        pltpu.SemaphoreType.DMA,
    ],
)
def _prop_kernel(s_hbm, src_hbm, dst_hbm, zrows_hbm, part_out,
                 acc_sp, src_v, dst_v, rows3_v, g0, g1, g2, s0, s1, s2):
    c = lax.axis_index("c")
    sid = lax.axis_index("s")
    wid = sid * NC + c
    gsem = (g0, g1, g2)
    ssem = (s0, s1, s2)

    pltpu.sync_copy(zrows_hbm, rows3_v.at[0])
    pltpu.sync_copy(src_hbm.at[wid], src_v)
    pltpu.sync_copy(dst_hbm.at[wid], dst_v)
    for k in range(KMAX_P):
        @pl.when(sid + k * NS < NCHUNK_P)
        def _(k=k):
            pltpu.sync_copy(rows3_v.at[0], acc_sp.at[pl.ds((sid + k * NS) * B, B)])

    plsc.subcore_barrier()

    def _g(bi, j):
        return pltpu.make_async_copy(
            s_hbm.at[src_v.at[pl.ds(bi * B, B)]], rows3_v.at[j], gsem[j]
        )

    def _s(bi, j):
        return pltpu.make_async_copy(
            rows3_v.at[j], acc_sp.at[dst_v.at[pl.ds(bi * B, B)]], ssem[j]
        )

    _g(0, 0).start()

    def body(gidx, carry):
        for j in range(3):
            b = 3 * gidx + j
            _g(b, j).wait()
            _s(b, j).start(add=True)

            @pl.when(b >= 2)
            def _(b=b, j=j):
                _s(b - 2, (j + 1) % 3).wait()

            _g(b + 1, (j + 1) % 3).start()
        return carry

    lax.fori_loop(0, (NB - 2) // 3, body, 0)
    _g(NB - 2, 0).wait()
    _s(NB - 2, 0).start(add=True)
    _s(NB - 4, 1).wait()
    _g(NB - 1, 1).start()
    _g(NB - 1, 1).wait()
    _s(NB - 1, 1).start(add=True)
    _s(NB - 3, 2).wait()
    _s(NB - 2, 0).wait()
    _s(NB - 1, 1).wait()
    plsc.subcore_barrier()

    def _stage(k, j):
        cidx = sid + k * NS
        return pltpu.make_async_copy(
            acc_sp.at[pl.ds(cidx * B, B)], rows3_v.at[j], gsem[j]
        )

    def _wr(k, j):
        cidx = sid + k * NS
        return pltpu.make_async_copy(
            rows3_v.at[j], part_out.at[c, pl.ds(cidx * B, B)], ssem[j]
        )

    for k in range(KMAX_P):
        if k >= 3:
            @pl.when(sid + (k - 3) * NS < NCHUNK_P)
            def _(k=k):
                _wr(k - 3, k % 3).wait()

        @pl.when(sid + k * NS < NCHUNK_P)
        def _(k=k):
            _stage(k, k % 3).start()
            _stage(k, k % 3).wait()
            _wr(k, k % 3).start()

    for k in range(max(0, KMAX_P - 3), KMAX_P):
        @pl.when(sid + k * NS < NCHUNK_P)
        def _(k=k):
            _wr(k, k % 3).wait()


def _prep_body(degp_ref, x_ref, s0_ref, dinv_ref, dsqrt_ref):
    deg = degp_ref[0] + degp_ref[1] + 1.0
    dis = lax.rsqrt(deg)
    s0_ref[...] = dis * x_ref[...]
    dinv_ref[...] = dis * dis
    dsqrt_ref[...] = deg * dis


_prep = pl.pallas_call(
    _prep_body,
    grid=(G,),
    in_specs=[
        pl.BlockSpec((NC, R, D), lambda i: (0, i, 0)),
        pl.BlockSpec((R, D), lambda i: (i, 0)),
    ],
    out_specs=[pl.BlockSpec((R, D), lambda i: (i, 0))] * 3,
    out_shape=[jax.ShapeDtypeStruct((N, D), jnp.float32)] * 3,
)


def _combine_body(part_ref, s_ref, dinv_ref, out_ref):
    out_ref[...] = dinv_ref[...] * (part_ref[0] + part_ref[1] + s_ref[...])


_combine = pl.pallas_call(
    _combine_body,
    grid=(G,),
    in_specs=[
        pl.BlockSpec((NC, R, D), lambda i: (0, i, 0)),
        pl.BlockSpec((R, D), lambda i: (i, 0)),
        pl.BlockSpec((R, D), lambda i: (i, 0)),
    ],
    out_specs=pl.BlockSpec((R, D), lambda i: (i, 0)),
    out_shape=jax.ShapeDtypeStruct((N, D), jnp.float32),
)


def _final_body(x_ref, s1_ref, s3_ref, s7_ref, dsq_ref, w_ref, b_ref, out_ref):
    dsq = dsq_ref[...]
    h1 = dsq * s1_ref[...]
    h3 = dsq * s3_ref[...]
    h7 = dsq * s7_ref[...]
    w = w_ref[...]
    w1 = w[:, 0:128]
    w2 = w[:, 128:256]
    w3 = w[:, 256:384]
    w4 = w[:, 384:512]
    w5 = w[:, 512:640]
    w6 = w[:, 640:768]
    wc = jnp.concatenate([w4, w1 - w4 + w5, w2 - w5 + w6, w3 - w6], axis=1)
    h = jnp.concatenate([x_ref[...], h1, h3, h7], axis=1)
    acc = lax.dot_general(h, wc, (((1,), (1,)), ((), ())),
                          preferred_element_type=jnp.float32)
    out_ref[...] = jnp.maximum(acc + b_ref[...], 0.0)


_final = pl.pallas_call(
    _final_body,
    grid=(G,),
    in_specs=[
        pl.BlockSpec((R, D), lambda i: (i, 0)),
        pl.BlockSpec((R, D), lambda i: (i, 0)),
        pl.BlockSpec((R, D), lambda i: (i, 0)),
        pl.BlockSpec((R, D), lambda i: (i, 0)),
        pl.BlockSpec((R, D), lambda i: (i, 0)),
        pl.BlockSpec((D, 6 * D), lambda i: (0, 0)),
        pl.BlockSpec((1, D), lambda i: (0, 0)),
    ],
    out_specs=pl.BlockSpec((R, D), lambda i: (i, 0)),
    out_shape=jax.ShapeDtypeStruct((N, D), jnp.float32),
)


def kernel(x, edge_index, W, b):
    src = edge_index[0].astype(jnp.int32).reshape(NW, EPT)
    dst = edge_index[1].astype(jnp.int32).reshape(NW, EPT)
    zrows = jnp.zeros((B, D), jnp.float32)
    ones_rows = jnp.ones((B, D), jnp.float32)

    degp = _deg_kernel(dst, zrows, ones_rows)
    s0, dinv, dsqrt = _prep(degp, x)

    s = s0
    snaps = {}
    for r in range(1, 8):
        part = _prop_kernel(s, src, dst, zrows)
        s = _combine(part, s, dinv)
        if r in (1, 3, 7):
            snaps[r] = s

    return _final(x, snaps[1], snaps[3], snaps[7], dsqrt, W, b.reshape(1, D))


# speedup vs baseline: 1.4273x; 1.4273x over previous
import functools

import jax
import jax.numpy as jnp
from jax import lax
from jax.experimental import pallas as pl
from jax.experimental.pallas import tpu as pltpu
from jax.experimental.pallas import tpu_sc as plsc

N = 10000
D = 128
E = 320000
NC = 2
NS = 16
NW = NC * NS
EPT = E // NW
B = 80
NB = EPT // B
NCHUNK_P = N // B
KMAX_P = -(-NCHUNK_P // NS)

R = 400
G = N // R

_mesh = plsc.VectorSubcoreMesh(
    core_axis_name="c", subcore_axis_name="s", num_cores=NC, num_subcores=NS
)


@functools.partial(
    pl.kernel,
    out_type=jax.ShapeDtypeStruct((NC, N, D), jnp.float32),
    mesh=_mesh,
    scratch_types=[
        pltpu.VMEM_SHARED((N, D), jnp.float32),
        pltpu.VMEM((EPT,), jnp.int32),
        pltpu.VMEM((2, B, D), jnp.float32),
        pltpu.SemaphoreType.DMA,
        pltpu.SemaphoreType.DMA,
    ],
)
def _deg_kernel(dst_hbm, zeros_hbm, ones_hbm, deg_out,
                deg_sp, dst_v, rows2_v, sem_a, sem_b):
    c = lax.axis_index("c")
    sid = lax.axis_index("s")
    wid = sid * NC + c
    zeros_v = rows2_v.at[0]
    ones_v = rows2_v.at[1]
    pltpu.sync_copy(zeros_hbm, zeros_v)
    pltpu.sync_copy(ones_hbm, ones_v)
    pltpu.sync_copy(dst_hbm.at[wid], dst_v)

    def _zc(k):
        cidx = sid + k * NS
        return pltpu.make_async_copy(
            zeros_v, deg_sp.at[pl.ds(cidx * B, B)], sem_a
        )

    for k in range(KMAX_P):
        @pl.when(sid + k * NS < NCHUNK_P)
        def _(k=k):
            _zc(k).start()

    for k in range(KMAX_P):
        @pl.when(sid + k * NS < NCHUNK_P)
        def _(k=k):
            _zc(k).wait()

    plsc.subcore_barrier()

    def _sc(bi):
        return pltpu.make_async_copy(
            ones_v, deg_sp.at[dst_v.at[pl.ds(bi * B, B)]], sem_a
        )

    def fire(bi, carry):
        _sc(bi).start(add=True)
        return carry

    def drain(bi, carry):
        _sc(bi).wait()
        return carry

    lax.fori_loop(0, NB, fire, 0)
    lax.fori_loop(0, NB, drain, 0)
    plsc.subcore_barrier()

    def _stage(k, j):
        cidx = sid + k * NS
        return pltpu.make_async_copy(
            deg_sp.at[pl.ds(cidx * B, B)], rows2_v.at[j], sem_a
        )

    def _wr(k, j):
        cidx = sid + k * NS
        return pltpu.make_async_copy(
            rows2_v.at[j], deg_out.at[c, pl.ds(cidx * B, B)], sem_b
        )

    for k in range(KMAX_P):
        if k >= 2:
            @pl.when(sid + (k - 2) * NS < NCHUNK_P)
            def _(k=k):
                _wr(k - 2, k % 2).wait()

        @pl.when(sid + k * NS < NCHUNK_P)
        def _(k=k):
            _stage(k, k % 2).start()
            _stage(k, k % 2).wait()
            _wr(k, k % 2).start()

    for k in range(max(0, KMAX_P - 2), KMAX_P):
        @pl.when(sid + k * NS < NCHUNK_P)
        def _(k=k):
            _wr(k, k % 2).wait()


@functools.partial(
    pl.kernel,
    out_type=jax.ShapeDtypeStruct((NC, N, D), jnp.float32),
    mesh=_mesh,
    scratch_types=[
        pltpu.VMEM_SHARED((N, D), jnp.float32),
        pltpu.VMEM((EPT,), jnp.int32),
        pltpu.VMEM((EPT,), jnp.int32),
        pltpu.VMEM((3, B, D), jnp.float32),
        pltpu.SemaphoreType.DMA,
        pltpu.SemaphoreType.DMA,
        pltpu.SemaphoreType.DMA,
        pltpu.SemaphoreType.DMA,
        pltpu.SemaphoreType.DMA,
        pltpu.SemaphoreType.DMA,
    ],
)
def _prop_kernel(s_hbm, src_hbm, dst_hbm, zrows_hbm, part_out,
                 acc_sp, src_v, dst_v, rows3_v, g0, g1, g2, s0, s1, s2):
    c = lax.axis_index("c")
    sid = lax.axis_index("s")
    wid = sid * NC + c
    gsem = (g0, g1, g2)
    ssem = (s0, s1, s2)

    pltpu.sync_copy(zrows_hbm, rows3_v.at[2])
    pltpu.sync_copy(src_hbm.at[wid], src_v)
    pltpu.sync_copy(dst_hbm.at[wid], dst_v)

    def _g(bi, j):
        return pltpu.make_async_copy(
            s_hbm.at[src_v.at[pl.ds(bi * B, B)]], rows3_v.at[j], gsem[j]
        )

    def _s(bi, j):
        return pltpu.make_async_copy(
            rows3_v.at[j], acc_sp.at[dst_v.at[pl.ds(bi * B, B)]], ssem[j]
        )

    _g(0, 0).start()
    _g(1, 1).start()
    for k in range(KMAX_P):
        @pl.when(sid + k * NS < NCHUNK_P)
        def _(k=k):
            pltpu.sync_copy(rows3_v.at[2], acc_sp.at[pl.ds((sid + k * NS) * B, B)])

    plsc.subcore_barrier()

    def body(gidx, carry):
        for j in range(3):
            b = 3 * gidx + j
            _g(b, j).wait()
            _s(b, j).start(add=True)
            _g(b + 2, (j + 2) % 3).start()
            _s(b, j).wait()
        return carry

    lax.fori_loop(0, (NB - 2) // 3, body, 0)
    _g(NB - 2, 0).wait()
    _s(NB - 2, 0).start(add=True)
    _s(NB - 2, 0).wait()
    _g(NB - 1, 1).wait()
    _s(NB - 1, 1).start(add=True)
    _s(NB - 1, 1).wait()
    plsc.subcore_barrier()

    def _stage(k, j):
        cidx = sid + k * NS
        return pltpu.make_async_copy(
            acc_sp.at[pl.ds(cidx * B, B)], rows3_v.at[j], gsem[j]
        )

    def _wr(k, j):
        cidx = sid + k * NS
        return pltpu.make_async_copy(
            rows3_v.at[j], part_out.at[c, pl.ds(cidx * B, B)], ssem[j]
        )

    for k in range(KMAX_P):
        if k >= 3:
            @pl.when(sid + (k - 3) * NS < NCHUNK_P)
            def _(k=k):
                _wr(k - 3, k % 3).wait()

        @pl.when(sid + k * NS < NCHUNK_P)
        def _(k=k):
            _stage(k, k % 3).start()
            _stage(k, k % 3).wait()
            _wr(k, k % 3).start()

    for k in range(max(0, KMAX_P - 3), KMAX_P):
        @pl.when(sid + k * NS < NCHUNK_P)
        def _(k=k):
            _wr(k, k % 3).wait()


def _prep_body(degp_ref, x_ref, s0_ref, dinv_ref, dsqrt_ref):
    deg = degp_ref[0] + degp_ref[1] + 1.0
    dis = lax.rsqrt(deg)
    s0_ref[...] = dis * x_ref[...]
    dinv_ref[...] = dis * dis
    dsqrt_ref[...] = deg * dis


_prep = pl.pallas_call(
    _prep_body,
    grid=(G,),
    in_specs=[
        pl.BlockSpec((NC, R, D), lambda i: (0, i, 0)),
        pl.BlockSpec((R, D), lambda i: (i, 0)),
    ],
    out_specs=[pl.BlockSpec((R, D), lambda i: (i, 0))] * 3,
    out_shape=[jax.ShapeDtypeStruct((N, D), jnp.float32)] * 3,
)


def _combine_body(part_ref, s_ref, dinv_ref, out_ref):
    out_ref[...] = dinv_ref[...] * (part_ref[0] + part_ref[1] + s_ref[...])


_combine = pl.pallas_call(
    _combine_body,
    grid=(G,),
    in_specs=[
        pl.BlockSpec((NC, R, D), lambda i: (0, i, 0)),
        pl.BlockSpec((R, D), lambda i: (i, 0)),
        pl.BlockSpec((R, D), lambda i: (i, 0)),
    ],
    out_specs=pl.BlockSpec((R, D), lambda i: (i, 0)),
    out_shape=jax.ShapeDtypeStruct((N, D), jnp.float32),
)


def _final_body(x_ref, s1_ref, s3_ref, s7_ref, dsq_ref, w_ref, b_ref, out_ref):
    dsq = dsq_ref[...]
    h1 = dsq * s1_ref[...]
    h3 = dsq * s3_ref[...]
    h7 = dsq * s7_ref[...]
    w = w_ref[...]
    w1 = w[:, 0:128]
    w2 = w[:, 128:256]
    w3 = w[:, 256:384]
    w4 = w[:, 384:512]
    w5 = w[:, 512:640]
    w6 = w[:, 640:768]
    wc = jnp.concatenate([w4, w1 - w4 + w5, w2 - w5 + w6, w3 - w6], axis=1)
    h = jnp.concatenate([x_ref[...], h1, h3, h7], axis=1)
    acc = lax.dot_general(h, wc, (((1,), (1,)), ((), ())),
                          preferred_element_type=jnp.float32)
    out_ref[...] = jnp.maximum(acc + b_ref[...], 0.0)


_final = pl.pallas_call(
    _final_body,
    grid=(G,),
    in_specs=[
        pl.BlockSpec((R, D), lambda i: (i, 0)),
        pl.BlockSpec((R, D), lambda i: (i, 0)),
        pl.BlockSpec((R, D), lambda i: (i, 0)),
        pl.BlockSpec((R, D), lambda i: (i, 0)),
        pl.BlockSpec((R, D), lambda i: (i, 0)),
        pl.BlockSpec((D, 6 * D), lambda i: (0, 0)),
        pl.BlockSpec((1, D), lambda i: (0, 0)),
    ],
    out_specs=pl.BlockSpec((R, D), lambda i: (i, 0)),
    out_shape=jax.ShapeDtypeStruct((N, D), jnp.float32),
)


def kernel(x, edge_index, W, b):
    src = edge_index[0].astype(jnp.int32).reshape(NW, EPT)
    dst = edge_index[1].astype(jnp.int32).reshape(NW, EPT)
    zrows = jnp.zeros((B, D), jnp.float32)
    ones_rows = jnp.ones((B, D), jnp.float32)

    degp = _deg_kernel(dst, zrows, ones_rows)
    s0, dinv, dsqrt = _prep(degp, x)

    s = s0
    snaps = {}
    for r in range(1, 8):
        part = _prop_kernel(s, src, dst, zrows)
        s = _combine(part, s, dinv)
        if r in (1, 3, 7):
            snaps[r] = s

    return _final(x, snaps[1], snaps[3], snaps[7], dsqrt, W, b.reshape(1, D))
